# baseline (speedup 1.0000x reference)
import jax
import jax.numpy as jnp
from jax import lax
from jax.experimental import pallas as pl
from jax.experimental.pallas import tpu as pltpu
from jax.experimental.pallas import tpu_sc as plsc

NX, NY, NZ = 256, 256, 1
C = 64
MAX_CAV = 4
N = 32768
UB = MAX_CAV
G = NX * NY

NC, NS, L = 2, 16, 16
NW = NC * NS
CELLS_PW = UB * G // NW
K = NX
NCH = CELLS_PW // K
B = 32
RING = 3
P = N // NS
VPC = P // L
KD = K + 16


def _body(coords_ref, pillar_ref, out_ref, cbuf, sbuf, s_sh, owner, gidx,
          cpos, table, ostage, nbref, csem, gsem, osem):
    sid = lax.axis_index("s")
    wid = sid * NC + lax.axis_index("c")
    cell_base = wid * CELLS_PW
    iota = jnp.arange(L, dtype=jnp.int32)
    zeros = jnp.zeros((L,), jnp.float32)

    neg1 = jnp.full((L,), -1, jnp.int32)

    @pl.loop(0, CELLS_PW // L, unroll=8)
    def _init(i):
        owner[pl.ds(i * L, L)] = neg1

    with jax.named_scope("phase0_index"):
        pltpu.sync_copy(coords_ref.at[pl.ds(sid * P, P)], cbuf)
        c0 = jnp.zeros((L,), jnp.int32)
        c2 = jnp.full((L,), 2, jnp.int32)
        c3 = jnp.full((L,), 3, jnp.int32)

        @pl.loop(0, VPC, unroll=4)
        def _idx(v):
            pvec = v * L + iota
            b = plsc.load_gather(cbuf, [pvec, c0])
            y = plsc.load_gather(cbuf, [pvec, c2])
            x = plsc.load_gather(cbuf, [pvec, c3])
            sbuf[pl.ds(v * L, L)] = b * G + y * NX + x

        pltpu.sync_copy(sbuf.at[pl.ds(0, P)], s_sh.at[pl.ds(sid * P, P)])
        plsc.subcore_barrier()

    def _fire_s(pc, par):
        pltpu.make_async_copy(s_sh.at[pl.ds(pc * P, P)],
                              sbuf.at[pl.ds(par * P, P)], csem).start()

    _fire_s(0, 0)

    with jax.named_scope("phase1_scan"):
        @pl.loop(0, NS)
        def _scan(pc):
            par = lax.rem(pc, 2)
            pltpu.make_async_copy(s_sh.at[pl.ds(0, P)],
                                  sbuf.at[pl.ds(0, P)], csem).wait()

            @pl.when(pc + 1 < NS)
            def _():
                _fire_s(pc + 1, 1 - par)

            cb = par * P

            @pl.loop(0, VPC, unroll=8)
            def _pts(v):
                svec = sbuf[pl.ds(cb + v * L, L)]
                rel = svec - cell_base
                m = (rel >= 0) & (rel < CELLS_PW)
                relc = jnp.clip(rel, 0, CELLS_PW - 1)
                ivec = pc * P + v * L + iota
                plsc.store_scatter(owner, [relc], ivec, mask=m)

    dumpK = jnp.full((L,), K, jnp.int32)

    def _build_and_fire_gathers(ch, par):
        gb = par * K

        @pl.loop(0, K // L, unroll=4)
        def _pre(u):
            spread = (ch * K + u * L + iota + wid * L) & (N - 1)
            gidx[pl.ds(gb + u * L, L)] = spread
            cpos[pl.ds(gb + u * L, L)] = dumpK

        @pl.loop(0, K // L, init_carry=jnp.int32(0))
        def _cmp(u, base):
            ov = owner[pl.ds(ch * K + u * L, L)]
            m = ov >= 0
            ones = m.astype(jnp.int32)
            pos = gb + base + plsc.cumsum(ones) - 1
            plsc.store_scatter(gidx, [pos], ov, mask=m)
            plsc.store_scatter(cpos, [pos], u * L + iota, mask=m)
            return base + jnp.sum(ones)

        n_occ = _cmp
        nb = (n_occ + (B - 1)) // B
        nbref[ch] = nb

        @pl.loop(0, nb)
        def _g(b):
            pltpu.async_copy(pillar_ref.at[gidx.at[pl.ds(gb + b * B, B)]],
                             table.at[pl.ds(gb + b * B, B)], gsem)

    def _drain_out():
        pltpu.make_async_copy(ostage.at[pl.ds(0, C), pl.ds(0, K)],
                              out_ref.at[0, :, 0, :], osem).wait()

    for r in range(RING - 1):
        _build_and_fire_gathers(r, r)

    @pl.loop(0, NCH)
    def _chunk(ch):
        par = lax.rem(ch, RING)
        ob = lax.rem(ch, 2) * C
        gb = par * K
        nb = nbref[ch]

        with jax.named_scope("wait_gathers"):
            @pl.loop(0, nb)
            def _w(b):
                pltpu.make_async_copy(
                    pillar_ref.at[gidx.at[pl.ds(0, B)]],
                    table.at[pl.ds(b * B, B)], gsem).wait()

        with jax.named_scope("build_fire_gathers"):
            @pl.when(ch + RING - 1 < NCH)
            def _():
                _build_and_fire_gathers(ch + RING - 1,
                                        lax.rem(ch + RING - 1, RING))

        with jax.named_scope("drain_out"):
            @pl.when(ch >= 2)
            def _():
                _drain_out()

        with jax.named_scope("zero_stage"):
            @pl.loop(0, C)
            def _z(c):
                @pl.loop(0, K // L, unroll=8)
                def _zv(v):
                    ostage[ob + c, pl.ds(v * L, L)] = zeros

        with jax.named_scope("transpose"):
            @pl.loop(0, nb)
            def _b(b):
                @pl.loop(0, B // L)
                def _v(v):
                    cpv = cpos[pl.ds(gb + b * B + v * L, L)]
                    rowvec = gb + b * B + v * L + iota

                    @pl.loop(0, C, unroll=4)
                    def _c(c):
                        cvec = jnp.full((L,), c, jnp.int32)
                        val = plsc.load_gather(table, [rowvec, cvec])
                        rvec = jnp.full((L,), ob + c, jnp.int32)
                        plsc.store_scatter(ostage, [rvec, cpv], val)

        cell0 = cell_base + ch * K
        bb = cell0 // G
        yrow = (cell0 - bb * G) // NX

        with jax.named_scope("fire_out"):
            pltpu.make_async_copy(
                ostage.at[pl.ds(ob, C), pl.ds(0, K)],
                out_ref.at[bb, :, yrow, :], osem).start()

    _drain_out()
    _drain_out()


@jax.jit
def _scatter_bev(coords, pillar):
    f = pl.kernel(
        _body,
        out_type=jax.ShapeDtypeStruct((UB, C, NY, NX), jnp.float32),
        mesh=plsc.VectorSubcoreMesh(core_axis_name="c", subcore_axis_name="s"),
        compiler_params=pltpu.CompilerParams(use_tc_tiling_on_sc=False,
                                             needs_layout_passes=False),
        scratch_types=[
            pltpu.VMEM((P, 4), jnp.int32),
            pltpu.VMEM((2 * P,), jnp.int32),
            pltpu.VMEM_SHARED((N,), jnp.int32),
            pltpu.VMEM((CELLS_PW,), jnp.int32),
            pltpu.VMEM((RING * K,), jnp.int32),
            pltpu.VMEM((RING * K,), jnp.int32),
            pltpu.VMEM((RING * K, C), jnp.float32),
            pltpu.VMEM((2 * C, KD), jnp.float32),
            pltpu.SMEM((NCH,), jnp.int32),
            pltpu.SemaphoreType.DMA,
            pltpu.SemaphoreType.DMA,
            pltpu.SemaphoreType.DMA,
        ],
    )
    return f(coords, pillar)


def kernel(voxel_coords, record_len, pillar_features):
    del record_len
    coords = voxel_coords.astype(jnp.int32)
    pillar = pillar_features.astype(jnp.float32)
    return _scatter_bev(coords, pillar)


# speedup vs baseline: 1.1096x; 1.1096x over previous
import jax
import jax.numpy as jnp
from jax import lax
from jax.experimental import pallas as pl
from jax.experimental.pallas import tpu as pltpu
from jax.experimental.pallas import tpu_sc as plsc

NX, NY, NZ = 256, 256, 1
C = 64
MAX_CAV = 4
N = 32768
UB = MAX_CAV
G = NX * NY

NC, NS, L = 2, 16, 16
NW = NC * NS
CELLS_PW = UB * G // NW
K = NX
NCH = CELLS_PW // K
B = 32
RING = 3
P = N // NS
VPC = P // L
KD = K + 16


def _body(coords_ref, pillar_ref, out_ref, cbuf, sbuf, s_sh, owner, gidx,
          cpos, table, ostage, nbref, csem, gsem, osem):
    sid = lax.axis_index("s")
    wid = sid * NC + lax.axis_index("c")
    cell_base = wid * CELLS_PW
    iota = jnp.arange(L, dtype=jnp.int32)
    zeros = jnp.zeros((L,), jnp.float32)

    neg1 = jnp.full((L,), -1, jnp.int32)

    @pl.loop(0, CELLS_PW // L, unroll=8)
    def _init(i):
        owner[pl.ds(i * L, L)] = neg1

    with jax.named_scope("phase0_index"):
        for r in range(3):
            pltpu.sync_copy(coords_ref.at[r, pl.ds(sid * P, P)],
                            cbuf.at[r])

        @pl.loop(0, VPC, unroll=4)
        def _idx(v):
            b = cbuf[0, pl.ds(v * L, L)]
            y = cbuf[1, pl.ds(v * L, L)]
            x = cbuf[2, pl.ds(v * L, L)]
            sbuf[pl.ds(v * L, L)] = b * G + y * NX + x

        pltpu.sync_copy(sbuf.at[pl.ds(0, P)], s_sh.at[pl.ds(sid * P, P)])
        plsc.subcore_barrier()

    def _fire_s(pc, par):
        pltpu.make_async_copy(s_sh.at[pl.ds(pc * P, P)],
                              sbuf.at[pl.ds(par * P, P)], csem).start()

    _fire_s(0, 0)

    with jax.named_scope("phase1_scan"):
        @pl.loop(0, NS)
        def _scan(pc):
            par = lax.rem(pc, 2)
            pltpu.make_async_copy(s_sh.at[pl.ds(0, P)],
                                  sbuf.at[pl.ds(0, P)], csem).wait()

            @pl.when(pc + 1 < NS)
            def _():
                _fire_s(pc + 1, 1 - par)

            cb = par * P

            @pl.loop(0, VPC, unroll=8)
            def _pts(v):
                svec = sbuf[pl.ds(cb + v * L, L)]
                rel = svec - cell_base
                m = (rel >= 0) & (rel < CELLS_PW)
                relc = jnp.clip(rel, 0, CELLS_PW - 1)
                ivec = pc * P + v * L + iota
                plsc.store_scatter(owner, [relc], ivec, mask=m)

    dumpK = jnp.full((L,), K, jnp.int32)

    def _build_and_fire_gathers(ch, par):
        gb = par * K

        @pl.loop(0, K // L, unroll=4)
        def _pre(u):
            spread = (ch * K + u * L + iota + wid * L) & (N - 1)
            gidx[pl.ds(gb + u * L, L)] = spread
            cpos[pl.ds(gb + u * L, L)] = dumpK

        @pl.loop(0, K // L, init_carry=jnp.int32(0))
        def _cmp(u, base):
            ov = owner[pl.ds(ch * K + u * L, L)]
            m = ov >= 0
            ones = m.astype(jnp.int32)
            pos = gb + base + plsc.cumsum(ones) - 1
            plsc.store_scatter(gidx, [pos], ov, mask=m)
            plsc.store_scatter(cpos, [pos], u * L + iota, mask=m)
            return base + jnp.sum(ones)

        n_occ = _cmp
        nb = (n_occ + (B - 1)) // B
        nbref[ch] = nb

        @pl.loop(0, nb)
        def _g(b):
            pltpu.async_copy(pillar_ref.at[gidx.at[pl.ds(gb + b * B, B)]],
                             table.at[pl.ds(gb + b * B, B)], gsem)

    def _drain_out():
        pltpu.make_async_copy(ostage.at[pl.ds(0, C), pl.ds(0, K)],
                              out_ref.at[0, :, 0, :], osem).wait()

    for r in range(RING - 1):
        _build_and_fire_gathers(r, r)

    @pl.loop(0, NCH)
    def _chunk(ch):
        par = lax.rem(ch, RING)
        ob = lax.rem(ch, 2) * C
        gb = par * K
        nb = nbref[ch]

        with jax.named_scope("wait_gathers"):
            @pl.loop(0, nb)
            def _w(b):
                pltpu.make_async_copy(
                    pillar_ref.at[gidx.at[pl.ds(0, B)]],
                    table.at[pl.ds(b * B, B)], gsem).wait()

        with jax.named_scope("build_fire_gathers"):
            @pl.when(ch + RING - 1 < NCH)
            def _():
                _build_and_fire_gathers(ch + RING - 1,
                                        lax.rem(ch + RING - 1, RING))

        with jax.named_scope("drain_out"):
            @pl.when(ch >= 2)
            def _():
                _drain_out()

        with jax.named_scope("zero_stage"):
            @pl.loop(0, C)
            def _z(c):
                @pl.loop(0, K // L, unroll=8)
                def _zv(v):
                    ostage[ob + c, pl.ds(v * L, L)] = zeros

        with jax.named_scope("transpose"):
            @pl.loop(0, nb)
            def _b(b):
                @pl.loop(0, B // L)
                def _v(v):
                    cpv = cpos[pl.ds(gb + b * B + v * L, L)]
                    rowvec = gb + b * B + v * L + iota

                    @pl.loop(0, C, unroll=4)
                    def _c(c):
                        cvec = jnp.full((L,), c, jnp.int32)
                        val = plsc.load_gather(table, [rowvec, cvec])
                        rvec = jnp.full((L,), ob + c, jnp.int32)
                        plsc.store_scatter(ostage, [rvec, cpv], val)

        cell0 = cell_base + ch * K
        bb = cell0 // G
        yrow = (cell0 - bb * G) // NX

        with jax.named_scope("fire_out"):
            pltpu.make_async_copy(
                ostage.at[pl.ds(ob, C), pl.ds(0, K)],
                out_ref.at[bb, :, yrow, :], osem).start()

    _drain_out()
    _drain_out()


@jax.jit
def _scatter_bev(coords, pillar):
    f = pl.kernel(
        _body,
        out_type=jax.ShapeDtypeStruct((UB, C, NY, NX), jnp.float32),
        mesh=plsc.VectorSubcoreMesh(core_axis_name="c", subcore_axis_name="s"),
        compiler_params=pltpu.CompilerParams(use_tc_tiling_on_sc=False,
                                             needs_layout_passes=False),
        scratch_types=[
            pltpu.VMEM((3, P), jnp.int32),
            pltpu.VMEM((2 * P,), jnp.int32),
            pltpu.VMEM_SHARED((N,), jnp.int32),
            pltpu.VMEM((CELLS_PW,), jnp.int32),
            pltpu.VMEM((RING * K,), jnp.int32),
            pltpu.VMEM((RING * K,), jnp.int32),
            pltpu.VMEM((RING * K, C), jnp.float32),
            pltpu.VMEM((2 * C, KD), jnp.float32),
            pltpu.SMEM((NCH,), jnp.int32),
            pltpu.SemaphoreType.DMA,
            pltpu.SemaphoreType.DMA,
            pltpu.SemaphoreType.DMA,
        ],
    )
    return f(coords, pillar)


def kernel(voxel_coords, record_len, pillar_features):
    del record_len
    coords = voxel_coords.astype(jnp.int32).T[jnp.array([0, 2, 3])]
    pillar = pillar_features.astype(jnp.float32)
    return _scatter_bev(coords, pillar)
